# baseline (speedup 1.0000x reference)
import functools

import jax
import jax.numpy as jnp
from jax import lax
from jax.experimental import pallas as pl
from jax.experimental.pallas import tpu as pltpu
from jax.experimental.pallas import tpu_sc as plsc

B = 1024
C = 100000
NW = 32
RPW = B // NW
SPAN = RPW * C
CH = 20000
NCH = SPAN // CH
NS = 2
NGRP = NCH // NS
NVEC = CH // 16


def _sc_body(x_hbm, lab_hbm, o_hbm, ibuf, obuf, labv, idxv, valv, isem, osem, fsem):
    wid = lax.axis_index("s") * 2 + lax.axis_index("c")
    g0 = wid * SPAN
    r0 = wid * RPW

    pltpu.sync_copy(lab_hbm.at[pl.ds(r0, RPW)], labv)
    lane = lax.iota(jnp.int32, 16)
    idxv[pl.ds(0, 16)] = (r0 + lane) * C + labv[pl.ds(0, 16)]
    idxv[pl.ds(16, 16)] = (r0 + 16 + lane) * C + labv[pl.ds(16, 16)]

    def start_in(j, slot):
        pltpu.make_async_copy(
            x_hbm.at[pl.ds(g0 + j * CH, CH)], ibuf.at[slot], isem.at[slot]
        ).start()

    def wait_in(j, slot):
        pltpu.make_async_copy(
            x_hbm.at[pl.ds(g0 + j * CH, CH)], ibuf.at[slot], isem.at[slot]
        ).wait()

    def start_out(j, slot):
        pltpu.make_async_copy(
            ibuf.at[slot], o_hbm.at[pl.ds(g0 + j * CH, CH)], osem.at[slot]
        ).start()

    def wait_out(j, slot):
        pltpu.make_async_copy(
            ibuf.at[slot], o_hbm.at[pl.ds(g0 + j * CH, CH)], osem.at[slot]
        ).wait()

    fix_gather = pltpu.make_async_copy(x_hbm.at[idxv], valv, fsem)
    fix_gather.start()

    for s in range(NS):
        start_in(s, s)

    def group(g, _):
        for s in range(NS):
            j = g * NS + s
            wait_in(j, s)

            @pl.when(g > 0)
            def _():
                wait_out(j - NS, s)


            start_out(j, s)

            @pl.when(g + 1 < NGRP)
            def _():
                start_in(j + NS, s)

        return 0

    lax.fori_loop(0, NGRP, group, 0)
    for s in range(NS):
        wait_out((NGRP - 1) * NS + s, s)

    fix_gather.wait()
    valv[pl.ds(0, 16)] = 256.0 * jnp.clip(valv[pl.ds(0, 16)], -1.0, 1.0)
    valv[pl.ds(16, 16)] = 256.0 * jnp.clip(valv[pl.ds(16, 16)], -1.0, 1.0)
    fix_scatter = pltpu.make_async_copy(valv, o_hbm.at[idxv], fsem)
    fix_scatter.start()
    fix_scatter.wait()


@functools.partial(jax.jit, static_argnums=())
def kernel(cos_theta, labels):
    b, c = cos_theta.shape
    x_flat = cos_theta.reshape(b * c)
    lab = labels.astype(jnp.int32)
    mesh = plsc.VectorSubcoreMesh(core_axis_name="c", subcore_axis_name="s")
    out = pl.kernel(
        _sc_body,
        out_type=jax.ShapeDtypeStruct((b * c,), jnp.float32),
        mesh=mesh,
        compiler_params=pltpu.CompilerParams(use_tc_tiling_on_sc=False),
        scratch_types=[
            pltpu.VMEM((NS, CH), jnp.float32),
            pltpu.VMEM((NS, CH), jnp.float32),
            pltpu.VMEM((RPW,), jnp.int32),
            pltpu.VMEM((RPW,), jnp.int32),
            pltpu.VMEM((RPW,), jnp.float32),
            pltpu.SemaphoreType.DMA((NS,)),
            pltpu.SemaphoreType.DMA((NS,)),
            pltpu.SemaphoreType.DMA,
        ],
    )(x_flat, lab)
    return out.reshape(b, c)


# speedup vs baseline: 2.0523x; 2.0523x over previous
import functools

import jax
import jax.numpy as jnp
from jax.experimental import pallas as pl

MARGIN = 0.25
GAMMA = 256.0
O_N = -MARGIN
DELTA_N = MARGIN

B = 1024
C = 100000

BLOCK_B = 256
BLOCK_C = 8192


def _body(lab_ref, x_ref, o_ref):
    j = pl.program_id(1)
    x = x_ref[...]
    cos = jnp.clip(x, -1.0, 1.0)
    alpha_n = jnp.maximum(cos - O_N, 0.0)
    logit_n = alpha_n * (cos - DELTA_N)
    col = jax.lax.broadcasted_iota(jnp.int32, x.shape, 1) + j * BLOCK_C
    is_label = col == lab_ref[...]
    o_ref[...] = jnp.where(is_label, cos, logit_n) * GAMMA


@functools.partial(jax.jit, static_argnums=())
def kernel(cos_theta, labels):
    b, c = cos_theta.shape
    lab2d = labels.astype(jnp.int32).reshape(b, 1)
    grid = (b // BLOCK_B, pl.cdiv(c, BLOCK_C))
    return pl.pallas_call(
        _body,
        grid=grid,
        in_specs=[
            pl.BlockSpec((BLOCK_B, 1), lambda i, j: (i, 0)),
            pl.BlockSpec((BLOCK_B, BLOCK_C), lambda i, j: (i, j)),
        ],
        out_specs=pl.BlockSpec((BLOCK_B, BLOCK_C), lambda i, j: (i, j)),
        out_shape=jax.ShapeDtypeStruct((b, c), jnp.float32),
    )(lab2d, cos_theta)
